# Initial kernel scaffold; baseline (speedup 1.0000x reference)
import jax
import jax.numpy as jnp
from jax.experimental import pallas as pl


def kernel(projected_region_features, segment2region_assignment, edge_index, cheb_W, cheb_b, ln_g, ln_b):
    raise NotImplementedError("write your pallas kernel here")


# speedup vs baseline: 8.6835x; 8.6835x over previous
import functools

import jax
import jax.numpy as jnp
from jax import lax
from jax.experimental import pallas as pl
from jax.experimental.pallas import tpu as pltpu
from jax.experimental.pallas import tpu_sc as plsc

NC, NS, LANES = 2, 16, 16
NW = NC * NS

N_SEG = 10000
R_DIM = 1000
D = 128
DH = D // 2
E = 320000
EB = 125
NB_TOTAL = E // EB
RPT = NB_TOTAL // NW
N_PAD = 10240
ROWS_ACC = N_PAD // NS
ZB = 128
DEG_W = 16
NROW_BLK = 1000
GRID_N = N_SEG // NROW_BLK


def _sc_deg_body(src_hbm, out_hbm, idx_v, buf_v, zb_v, sem, acc):
    del sem
    c = lax.axis_index("c")
    s = lax.axis_index("s")
    wid = c * NS + s

    z = jnp.zeros((LANES,), jnp.float32)

    def fill_zero(i, _):
        zb_v[i, :] = z
        return 0

    lax.fori_loop(0, ZB, fill_zero, 0)
    for q in range(ROWS_ACC // ZB):
        pltpu.sync_copy(zb_v, acc.at[pl.ds(s * ROWS_ACC + q * ZB, ZB)])

    o = jnp.ones((LANES,), jnp.float32)

    def fill_one(i, _):
        buf_v[i, :] = o
        return 0

    lax.fori_loop(0, EB, fill_one, 0)
    pltpu.sync_copy(src_hbm.at[pl.ds(wid * RPT, RPT)], idx_v)
    plsc.subcore_barrier()

    def step(j, _):
        pltpu.sync_copy(buf_v, acc.at[idx_v.at[j]], add=True)
        return 0

    lax.fori_loop(0, RPT, step, 0)
    plsc.subcore_barrier()
    pltpu.sync_copy(acc.at[pl.ds(s * ROWS_ACC, ROWS_ACC)],
                    out_hbm.at[c, pl.ds(s * ROWS_ACC, ROWS_ACC)])


_sc_deg = pl.kernel(
    _sc_deg_body,
    out_type=jax.ShapeDtypeStruct((NC, N_PAD, DEG_W), jnp.float32),
    mesh=plsc.VectorSubcoreMesh(core_axis_name="c", subcore_axis_name="s"),
    scratch_types=[
        pltpu.VMEM((RPT, EB), jnp.int32),
        pltpu.VMEM((EB, DEG_W), jnp.float32),
        pltpu.VMEM((ZB, DEG_W), jnp.float32),
        pltpu.SemaphoreType.DMA,
        pltpu.VMEM_SHARED((N_PAD, DEG_W), jnp.float32),
    ],
    compiler_params=pltpu.CompilerParams(use_tc_tiling_on_sc=False),
)


def _sc_matvec_body(glo_hbm, ghi_hbm, src_hbm, dst_hbm, out_hbm,
                    isrc_v, idst_v, rows_v, zb_v, sem, acc):
    c = lax.axis_index("c")
    s = lax.axis_index("s")
    wid = c * NS + s

    z = jnp.zeros((LANES,), jnp.float32)

    def fill_zero(i, _):
        for l in range(DH // LANES):
            zb_v[i, pl.ds(l * LANES, LANES)] = z
        return 0

    lax.fori_loop(0, ZB, fill_zero, 0)
    pltpu.sync_copy(src_hbm.at[pl.ds(wid * RPT, RPT)], isrc_v)
    pltpu.sync_copy(dst_hbm.at[pl.ds(wid * RPT, RPT)], idst_v)

    for h, g_hbm in enumerate((glo_hbm, ghi_hbm)):
        for q in range(ROWS_ACC // ZB):
            pltpu.sync_copy(zb_v, acc.at[pl.ds(s * ROWS_ACC + q * ZB, ZB)])
        plsc.subcore_barrier()

        def step(j, _):
            pltpu.async_copy(g_hbm.at[isrc_v.at[j]], rows_v, sem).wait()
            pltpu.sync_copy(rows_v, acc.at[idst_v.at[j]], add=True)
            return 0

        lax.fori_loop(0, RPT, step, 0)
        plsc.subcore_barrier()
        pltpu.sync_copy(acc.at[pl.ds(s * ROWS_ACC, ROWS_ACC)],
                        out_hbm.at[c, h, pl.ds(s * ROWS_ACC, ROWS_ACC)])


_sc_matvec = pl.kernel(
    _sc_matvec_body,
    out_type=jax.ShapeDtypeStruct((NC, 2, N_PAD, DH), jnp.float32),
    mesh=plsc.VectorSubcoreMesh(core_axis_name="c", subcore_axis_name="s"),
    scratch_types=[
        pltpu.VMEM((RPT, EB), jnp.int32),
        pltpu.VMEM((RPT, EB), jnp.int32),
        pltpu.VMEM((EB, DH), jnp.float32),
        pltpu.VMEM((ZB, DH), jnp.float32),
        pltpu.SemaphoreType.DMA,
        pltpu.VMEM_SHARED((N_PAD, DH), jnp.float32),
    ],
    compiler_params=pltpu.CompilerParams(use_tc_tiling_on_sc=False),
)


def _dis_body(p_ref, dis_ref):
    d = p_ref[0, 0:N_SEG, 0:1] + p_ref[1, 0:N_SEG, 0:1]
    dis_ref[...] = jnp.where(d > 0.0, lax.rsqrt(jnp.maximum(d, 1.0)), 0.0)


_t_dis = pl.pallas_call(
    _dis_body,
    out_shape=jax.ShapeDtypeStruct((N_SEG, 1), jnp.float32),
)


def _mm_body(s_ref, p_ref, dis_ref, raw_ref, glo_ref, ghi_ref):
    r = jnp.dot(s_ref[...], p_ref[...], preferred_element_type=jnp.float32)
    raw_ref[...] = r
    g = r * dis_ref[...]
    glo_ref[...] = g[:, :DH]
    ghi_ref[...] = g[:, DH:]


_half_spec = pl.BlockSpec((NROW_BLK, DH), lambda i: (i, 0))
_row_spec = pl.BlockSpec((NROW_BLK, D), lambda i: (i, 0))

_t_matmul = pl.pallas_call(
    _mm_body,
    grid=(GRID_N,),
    in_specs=[
        pl.BlockSpec((NROW_BLK, R_DIM), lambda i: (i, 0)),
        pl.BlockSpec((R_DIM, D), lambda i: (0, 0)),
        pl.BlockSpec((NROW_BLK, 1), lambda i: (i, 0)),
    ],
    out_specs=[_row_spec, _half_spec, _half_spec],
    out_shape=[
        jax.ShapeDtypeStruct((N_SEG, D), jnp.float32),
        jax.ShapeDtypeStruct((N_SEG, DH), jnp.float32),
        jax.ShapeDtypeStruct((N_SEG, DH), jnp.float32),
    ],
)


def _comb_body(alpha, beta, p_ref, prev_ref, dis_ref, tx_ref, glo_ref, ghi_ref):
    lo = p_ref[0, 0] + p_ref[1, 0]
    hi = p_ref[0, 1] + p_ref[1, 1]
    sm = jnp.concatenate([lo, hi], axis=-1)
    dis = dis_ref[...]
    tx = alpha * (-dis) * sm + beta * prev_ref[...]
    tx_ref[...] = tx
    g = dis * tx
    glo_ref[...] = g[:, :DH]
    ghi_ref[...] = g[:, DH:]


def _make_comb(alpha, beta):
    return pl.pallas_call(
        functools.partial(_comb_body, alpha, beta),
        grid=(GRID_N,),
        in_specs=[
            pl.BlockSpec((NC, 2, NROW_BLK, DH), lambda i: (0, 0, i, 0)),
            _row_spec,
            pl.BlockSpec((NROW_BLK, 1), lambda i: (i, 0)),
        ],
        out_specs=[_row_spec, _half_spec, _half_spec],
        out_shape=[
            jax.ShapeDtypeStruct((N_SEG, D), jnp.float32),
            jax.ShapeDtypeStruct((N_SEG, DH), jnp.float32),
            jax.ShapeDtypeStruct((N_SEG, DH), jnp.float32),
        ],
    )


_t_comb_first = _make_comb(1.0, 0.0)
_t_comb_rec = _make_comb(2.0, -1.0)


def _final_body(x_ref, t1_ref, t2_ref, t3_ref, t4_ref, w_ref, b_ref,
                lng_ref, lnb_ref, o_ref):
    acc = jnp.dot(x_ref[...], w_ref[0], preferred_element_type=jnp.float32)
    for k, tr in enumerate((t1_ref, t2_ref, t3_ref, t4_ref)):
        acc = acc + jnp.dot(tr[...], w_ref[k + 1],
                            preferred_element_type=jnp.float32)
    a = acc + b_ref[...]
    ge = 0.5 * a * (1.0 + lax.erf(a * 0.7071067811865476))
    mu = jnp.mean(ge, axis=-1, keepdims=True)
    var = jnp.mean((ge - mu) ** 2, axis=-1, keepdims=True)
    o_ref[...] = (ge - mu) * lax.rsqrt(var + 1e-5) * lng_ref[...] + lnb_ref[...]


_vec_spec = pl.BlockSpec((1, D), lambda i: (0, 0))

_t_final = pl.pallas_call(
    _final_body,
    grid=(GRID_N,),
    in_specs=[
        _row_spec, _row_spec, _row_spec, _row_spec, _row_spec,
        pl.BlockSpec((5, D, D), lambda i: (0, 0, 0)),
        _vec_spec, _vec_spec, _vec_spec,
    ],
    out_specs=_row_spec,
    out_shape=jax.ShapeDtypeStruct((N_SEG, D), jnp.float32),
)


def kernel(projected_region_features, segment2region_assignment, edge_index,
           cheb_W, cheb_b, ln_g, ln_b):
    srcr = edge_index[0].reshape(NB_TOTAL, EB)
    dstr = edge_index[1].reshape(NB_TOTAL, EB)

    degp = _sc_deg(srcr)
    dis = _t_dis(degp)
    raw, glo, ghi = _t_matmul(segment2region_assignment,
                              projected_region_features, dis)

    p1 = _sc_matvec(glo, ghi, srcr, dstr)
    tx1, glo, ghi = _t_comb_first(p1, raw, dis)
    p2 = _sc_matvec(glo, ghi, srcr, dstr)
    tx2, glo, ghi = _t_comb_rec(p2, raw, dis)
    p3 = _sc_matvec(glo, ghi, srcr, dstr)
    tx3, glo, ghi = _t_comb_rec(p3, tx1, dis)
    p4 = _sc_matvec(glo, ghi, srcr, dstr)
    tx4, _, _ = _t_comb_rec(p4, tx2, dis)

    out = _t_final(raw, tx1, tx2, tx3, tx4, cheb_W,
                   cheb_b.reshape(1, D), ln_g.reshape(1, D),
                   ln_b.reshape(1, D))
    return (out, raw)


# speedup vs baseline: 11.7799x; 1.3566x over previous
import functools

import jax
import jax.numpy as jnp
from jax import lax
from jax.experimental import pallas as pl
from jax.experimental.pallas import tpu as pltpu
from jax.experimental.pallas import tpu_sc as plsc

NC, NS, LANES = 2, 16, 16
NW = NC * NS

N_SEG = 10000
R_DIM = 1000
D = 128
DH = D // 2
E = 320000
EB = 125
NB_TOTAL = E // EB
RPT = NB_TOTAL // NW
KG = 5
N_PAD = 10240
ROWS_ACC = N_PAD // NS
ZB = 128
DEG_W = 16
NROW_BLK = 1000
GRID_N = N_SEG // NROW_BLK


def _sc_deg_body(src_hbm, out_hbm, idx_v, buf_v, zb_v, sem, acc):
    c = lax.axis_index("c")
    s = lax.axis_index("s")
    wid = c * NS + s

    z = jnp.zeros((LANES,), jnp.float32)

    def fill_zero(i, _):
        zb_v[i, :] = z
        return 0

    lax.fori_loop(0, ZB, fill_zero, 0)
    for q in range(ROWS_ACC // ZB):
        pltpu.sync_copy(zb_v, acc.at[pl.ds(s * ROWS_ACC + q * ZB, ZB)])

    o = jnp.ones((LANES,), jnp.float32)

    def fill_one(i, _):
        buf_v[i, :] = o
        return 0

    lax.fori_loop(0, EB, fill_one, 0)
    pltpu.sync_copy(src_hbm.at[pl.ds(wid * RPT, RPT)], idx_v)
    plsc.subcore_barrier()

    def step(t, _):
        descs = [
            pltpu.async_copy(buf_v, acc.at[idx_v.at[t * KG + i]], sem,
                             add=True)
            for i in range(KG)
        ]
        for d in descs:
            d.wait()
        return 0

    lax.fori_loop(0, RPT // KG, step, 0)
    plsc.subcore_barrier()
    pltpu.sync_copy(acc.at[pl.ds(s * ROWS_ACC, ROWS_ACC)],
                    out_hbm.at[c, pl.ds(s * ROWS_ACC, ROWS_ACC)])


_sc_deg = pl.kernel(
    _sc_deg_body,
    out_type=jax.ShapeDtypeStruct((NC, N_PAD, DEG_W), jnp.float32),
    mesh=plsc.VectorSubcoreMesh(core_axis_name="c", subcore_axis_name="s"),
    scratch_types=[
        pltpu.VMEM((RPT, EB), jnp.int32),
        pltpu.VMEM((EB, DEG_W), jnp.float32),
        pltpu.VMEM((ZB, DEG_W), jnp.float32),
        pltpu.SemaphoreType.DMA,
        pltpu.VMEM_SHARED((N_PAD, DEG_W), jnp.float32),
    ],
    compiler_params=pltpu.CompilerParams(use_tc_tiling_on_sc=False),
)


def _sc_matvec_body(glo_hbm, ghi_hbm, src_hbm, dst_hbm, out_hbm,
                    isrc_v, idst_v, rows_v, zb_v, gsem, ssem, acc):
    c = lax.axis_index("c")
    s = lax.axis_index("s")
    wid = c * NS + s

    z = jnp.zeros((LANES,), jnp.float32)

    def fill_zero(i, _):
        for l in range(DH // LANES):
            zb_v[i, pl.ds(l * LANES, LANES)] = z
        return 0

    lax.fori_loop(0, ZB, fill_zero, 0)
    pltpu.sync_copy(src_hbm.at[pl.ds(wid * RPT, RPT)], isrc_v)
    pltpu.sync_copy(dst_hbm.at[pl.ds(wid * RPT, RPT)], idst_v)

    for h, g_hbm in enumerate((glo_hbm, ghi_hbm)):
        for q in range(ROWS_ACC // ZB):
            pltpu.sync_copy(zb_v, acc.at[pl.ds(s * ROWS_ACC + q * ZB, ZB)])
        plsc.subcore_barrier()

        def step(t, _):
            gd = [
                pltpu.async_copy(g_hbm.at[isrc_v.at[t * KG + i]],
                                 rows_v.at[i], gsem)
                for i in range(KG)
            ]
            for d in gd:
                d.wait()
            sd = [
                pltpu.async_copy(rows_v.at[i], acc.at[idst_v.at[t * KG + i]],
                                 ssem, add=True)
                for i in range(KG)
            ]
            for d in sd:
                d.wait()
            return 0

        lax.fori_loop(0, RPT // KG, step, 0)
        plsc.subcore_barrier()
        pltpu.sync_copy(acc.at[pl.ds(s * ROWS_ACC, ROWS_ACC)],
                        out_hbm.at[c, h, pl.ds(s * ROWS_ACC, ROWS_ACC)])


_sc_matvec = pl.kernel(
    _sc_matvec_body,
    out_type=jax.ShapeDtypeStruct((NC, 2, N_PAD, DH), jnp.float32),
    mesh=plsc.VectorSubcoreMesh(core_axis_name="c", subcore_axis_name="s"),
    scratch_types=[
        pltpu.VMEM((RPT, EB), jnp.int32),
        pltpu.VMEM((RPT, EB), jnp.int32),
        pltpu.VMEM((KG, EB, DH), jnp.float32),
        pltpu.VMEM((ZB, DH), jnp.float32),
        pltpu.SemaphoreType.DMA,
        pltpu.SemaphoreType.DMA,
        pltpu.VMEM_SHARED((N_PAD, DH), jnp.float32),
    ],
    compiler_params=pltpu.CompilerParams(use_tc_tiling_on_sc=False),
)


def _dis_body(p_ref, dis_ref):
    d = p_ref[0, 0:N_SEG, 0:1] + p_ref[1, 0:N_SEG, 0:1]
    dis_ref[...] = jnp.where(d > 0.0, lax.rsqrt(jnp.maximum(d, 1.0)), 0.0)


_t_dis = pl.pallas_call(
    _dis_body,
    out_shape=jax.ShapeDtypeStruct((N_SEG, 1), jnp.float32),
)


def _mm_body(s_ref, p_ref, dis_ref, raw_ref, glo_ref, ghi_ref):
    r = jnp.dot(s_ref[...], p_ref[...], preferred_element_type=jnp.float32)
    raw_ref[...] = r
    g = r * dis_ref[...]
    glo_ref[...] = g[:, :DH]
    ghi_ref[...] = g[:, DH:]


_half_spec = pl.BlockSpec((NROW_BLK, DH), lambda i: (i, 0))
_row_spec = pl.BlockSpec((NROW_BLK, D), lambda i: (i, 0))

_t_matmul = pl.pallas_call(
    _mm_body,
    grid=(GRID_N,),
    in_specs=[
        pl.BlockSpec((NROW_BLK, R_DIM), lambda i: (i, 0)),
        pl.BlockSpec((R_DIM, D), lambda i: (0, 0)),
        pl.BlockSpec((NROW_BLK, 1), lambda i: (i, 0)),
    ],
    out_specs=[_row_spec, _half_spec, _half_spec],
    out_shape=[
        jax.ShapeDtypeStruct((N_SEG, D), jnp.float32),
        jax.ShapeDtypeStruct((N_SEG, DH), jnp.float32),
        jax.ShapeDtypeStruct((N_SEG, DH), jnp.float32),
    ],
)


def _comb_body(alpha, beta, p_ref, prev_ref, dis_ref, tx_ref, glo_ref, ghi_ref):
    lo = p_ref[0, 0] + p_ref[1, 0]
    hi = p_ref[0, 1] + p_ref[1, 1]
    sm = jnp.concatenate([lo, hi], axis=-1)
    dis = dis_ref[...]
    tx = alpha * (-dis) * sm + beta * prev_ref[...]
    tx_ref[...] = tx
    g = dis * tx
    glo_ref[...] = g[:, :DH]
    ghi_ref[...] = g[:, DH:]


def _make_comb(alpha, beta):
    return pl.pallas_call(
        functools.partial(_comb_body, alpha, beta),
        grid=(GRID_N,),
        in_specs=[
            pl.BlockSpec((NC, 2, NROW_BLK, DH), lambda i: (0, 0, i, 0)),
            _row_spec,
            pl.BlockSpec((NROW_BLK, 1), lambda i: (i, 0)),
        ],
        out_specs=[_row_spec, _half_spec, _half_spec],
        out_shape=[
            jax.ShapeDtypeStruct((N_SEG, D), jnp.float32),
            jax.ShapeDtypeStruct((N_SEG, DH), jnp.float32),
            jax.ShapeDtypeStruct((N_SEG, DH), jnp.float32),
        ],
    )


_t_comb_first = _make_comb(1.0, 0.0)
_t_comb_rec = _make_comb(2.0, -1.0)


def _final_body(x_ref, t1_ref, t2_ref, t3_ref, t4_ref, w_ref, b_ref,
                lng_ref, lnb_ref, o_ref):
    acc = jnp.dot(x_ref[...], w_ref[0], preferred_element_type=jnp.float32)
    for k, tr in enumerate((t1_ref, t2_ref, t3_ref, t4_ref)):
        acc = acc + jnp.dot(tr[...], w_ref[k + 1],
                            preferred_element_type=jnp.float32)
    a = acc + b_ref[...]
    ge = 0.5 * a * (1.0 + lax.erf(a * 0.7071067811865476))
    mu = jnp.mean(ge, axis=-1, keepdims=True)
    var = jnp.mean((ge - mu) ** 2, axis=-1, keepdims=True)
    o_ref[...] = (ge - mu) * lax.rsqrt(var + 1e-5) * lng_ref[...] + lnb_ref[...]


_vec_spec = pl.BlockSpec((1, D), lambda i: (0, 0))

_t_final = pl.pallas_call(
    _final_body,
    grid=(GRID_N,),
    in_specs=[
        _row_spec, _row_spec, _row_spec, _row_spec, _row_spec,
        pl.BlockSpec((5, D, D), lambda i: (0, 0, 0)),
        _vec_spec, _vec_spec, _vec_spec,
    ],
    out_specs=_row_spec,
    out_shape=jax.ShapeDtypeStruct((N_SEG, D), jnp.float32),
)


def kernel(projected_region_features, segment2region_assignment, edge_index,
           cheb_W, cheb_b, ln_g, ln_b):
    srcr = edge_index[0].reshape(NB_TOTAL, EB)
    dstr = edge_index[1].reshape(NB_TOTAL, EB)

    degp = _sc_deg(srcr)
    dis = _t_dis(degp)
    raw, glo, ghi = _t_matmul(segment2region_assignment,
                              projected_region_features, dis)

    p1 = _sc_matvec(glo, ghi, srcr, dstr)
    tx1, glo, ghi = _t_comb_first(p1, raw, dis)
    p2 = _sc_matvec(glo, ghi, srcr, dstr)
    tx2, glo, ghi = _t_comb_rec(p2, raw, dis)
    p3 = _sc_matvec(glo, ghi, srcr, dstr)
    tx3, glo, ghi = _t_comb_rec(p3, tx1, dis)
    p4 = _sc_matvec(glo, ghi, srcr, dstr)
    tx4, _, _ = _t_comb_rec(p4, tx2, dis)

    out = _t_final(raw, tx1, tx2, tx3, tx4, cheb_W,
                   cheb_b.reshape(1, D), ln_g.reshape(1, D),
                   ln_b.reshape(1, D))
    return (out, raw)
